# baseline (speedup 1.0000x reference)
import functools

import jax
import jax.numpy as jnp
from jax import lax
from jax.experimental import pallas as pl
from jax.experimental.pallas import tpu as pltpu
from jax.experimental.pallas import tpu_sc as plsc

DD, HH, WW = 64, 256, 256
NN = 20000
C = 128
TAPS = 9

NC, NS = 2, 16
NW = NC * NS
NPAD = 20480
RPW = NPAD // NW
CHUNK = 128
NCHUNK = RPW // CHUNK

G = DD * HH * WW
GROWS = 16512
GPAD = GROWS * 256
SENT = NN

_OFF_Z1X = [(dz, 0, dx) for dz in (-1, 0, 1) for dx in (-1, 0, 1)]
_OFF_1YX = [(0, dy, dx) for dy in (-1, 0, 1) for dx in (-1, 0, 1)]
_OFF_ALL = _OFF_Z1X[:3] + _OFF_Z1X[3:6] + _OFF_Z1X[6:] + _OFF_1YX[:3] + _OFF_1YX[6:]
_TAP_DEST = (
    [(i, None) for i in range(3)]
    + [(i, i) for i in range(3, 6)]
    + [(i, None) for i in range(6, 9)]
    + [(None, j) for j in range(3)]
    + [(None, j) for j in range(6, 9)]
)

import functools as _ft


@_ft.lru_cache(maxsize=None)
def _sc_mesh():
    return plsc.VectorSubcoreMesh(
        core_axis_name="c", subcore_axis_name="s",
        num_cores=NC, num_subcores=NS)


def _wid():
    return lax.axis_index("s") * NC + lax.axis_index("c")


def _mm_body(x_ref, wa_ref, wb_ref, oa_ref, ob_ref):
    x = x_ref[...].astype(jnp.bfloat16)
    oa_ref[0] = jnp.dot(x, wa_ref[0].astype(jnp.bfloat16),
                        preferred_element_type=jnp.float32)
    ob_ref[0] = jnp.dot(x, wb_ref[0].astype(jnp.bfloat16),
                        preferred_element_type=jnp.float32)


def _stage0_mm(f_pad, w1, w2):
    bm = 256
    return pl.pallas_call(
        _mm_body,
        grid=(NPAD // bm, TAPS),
        in_specs=[
            pl.BlockSpec((bm, C), lambda i, j: (i, 0)),
            pl.BlockSpec((1, C, C), lambda i, j: (j, 0, 0)),
            pl.BlockSpec((1, C, C), lambda i, j: (j, 0, 0)),
        ],
        out_specs=[
            pl.BlockSpec((1, bm, C), lambda i, j: (j, i, 0)),
            pl.BlockSpec((1, bm, C), lambda i, j: (j, i, 0)),
        ],
        out_shape=[
            jax.ShapeDtypeStruct((TAPS, NPAD, C), jnp.float32),
            jax.ShapeDtypeStruct((TAPS, NPAD, C), jnp.float32),
        ],
    )(f_pad, w1, w2)


def _stats_body(h1_ref, h2_ref, out_ref):
    @pl.when(pl.program_id(0) == 0)
    def _init():
        out_ref[...] = jnp.zeros((8, C), jnp.float32)

    h1 = h1_ref[...]
    h2 = h2_ref[...]
    upd = jnp.concatenate([
        jnp.sum(h1, axis=0)[None, :],
        jnp.sum(h1 * h1, axis=0)[None, :],
        jnp.sum(h2, axis=0)[None, :],
        jnp.sum(h2 * h2, axis=0)[None, :],
        jnp.zeros((4, C), jnp.float32),
    ], axis=0)
    out_ref[...] += upd


def _bn_stats(h1, h2):
    bm = 2048
    return pl.pallas_call(
        _stats_body,
        grid=(NPAD // bm,),
        in_specs=[
            pl.BlockSpec((bm, C), lambda i: (i, 0)),
            pl.BlockSpec((bm, C), lambda i: (i, 0)),
        ],
        out_specs=pl.BlockSpec((8, C), lambda i: (0, 0)),
        out_shape=jax.ShapeDtypeStruct((8, C), jnp.float32),
    )(h1, h2)


def _mm2_body(h1_ref, h2_ref, st_ref, g1_ref, b1_ref, g2_ref, b2_ref,
              w1_ref, w2_ref, oa_ref, ob_ref):
    bm = h1_ref.shape[0]
    eps = 1e-5
    n = jnp.float32(NN)
    row = pl.program_id(0) * bm + lax.broadcasted_iota(jnp.int32, (bm, 1), 0)
    live = (row < NN).astype(jnp.float32)

    def bn(h, srow, g, b):
        m = st_ref[srow] / n
        v = st_ref[srow + 1] / n - m * m
        s = g * lax.rsqrt(v + eps)
        t = b - m * s
        return (h * s[None, :] + t[None, :]) * live

    hb1 = bn(h1_ref[...], 0, g1_ref[...], b1_ref[...])
    hb2 = bn(h2_ref[...], 2, g2_ref[...], b2_ref[...])
    oa_ref[0] = jnp.dot(hb1.astype(jnp.bfloat16),
                        w1_ref[0].astype(jnp.bfloat16),
                        preferred_element_type=jnp.float32)
    ob_ref[0] = jnp.dot(hb2.astype(jnp.bfloat16),
                        w2_ref[0].astype(jnp.bfloat16),
                        preferred_element_type=jnp.float32)


def _stage2_mm(h1, h2, stats, g1, b1, g2, b2, w1, w2):
    bm = 256
    vec = lambda: pl.BlockSpec((C,), lambda i, j: (0,))
    return pl.pallas_call(
        _mm2_body,
        grid=(NPAD // bm, TAPS),
        in_specs=[
            pl.BlockSpec((bm, C), lambda i, j: (i, 0)),
            pl.BlockSpec((bm, C), lambda i, j: (i, 0)),
            pl.BlockSpec((8, C), lambda i, j: (0, 0)),
            vec(), vec(), vec(), vec(),
            pl.BlockSpec((1, C, C), lambda i, j: (j, 0, 0)),
            pl.BlockSpec((1, C, C), lambda i, j: (j, 0, 0)),
        ],
        out_specs=[
            pl.BlockSpec((1, bm, C), lambda i, j: (j, i, 0)),
            pl.BlockSpec((1, bm, C), lambda i, j: (j, i, 0)),
        ],
        out_shape=[
            jax.ShapeDtypeStruct((TAPS, NPAD, C), jnp.float32),
            jax.ShapeDtypeStruct((TAPS, NPAD, C), jnp.float32),
        ],
    )(h1, h2, stats, g1, b1, g2, b2, w1, w2)


_SLAB = GPAD // NS
_SLAB_CH = 16512
_VROWS = 16
_NACT = (NPAD // 128) // _VROWS


@_ft.lru_cache(maxsize=None)
def _make_sc_scatter():
  return functools.partial(
    pl.kernel,
    out_type=jax.ShapeDtypeStruct((NC * GPAD,), jnp.int32),
    mesh=_sc_mesh(),
    scratch_types=[
        pltpu.VMEM((128,), jnp.int32),
        pltpu.VMEM((128,), jnp.int32),
        pltpu.VMEM((_SLAB_CH,), jnp.int32),
        pltpu.SemaphoreType.DMA,
    ],
  )(_sc_scatter_body)


def _sc_scatter_body(lin_hbm, rid_hbm, out_hbm, lin_v, rid_v, fill, sem):
    c = lax.axis_index("c")
    s = lax.axis_index("s")
    coff = c * GPAD

    def fbody(k, _):
        fill[pl.ds(k * 16, 16)] = jnp.full((16,), SENT, jnp.int32)
        return 0

    lax.fori_loop(0, _SLAB_CH // 16, fbody, 0, unroll=8)
    gb = coff + s * _SLAB
    for j in range(_SLAB // _SLAB_CH):
        pltpu.sync_copy(fill, out_hbm.at[pl.ds(gb + j * _SLAB_CH, _SLAB_CH)])
    plsc.subcore_barrier()

    @pl.when(s < _NACT)
    def _do_scatter():
        vb = s * (_VROWS * 128)
        for j in range(_VROWS):
            off = vb + j * 128
            pltpu.sync_copy(lin_hbm.at[pl.ds(off, 128)], lin_v)
            pltpu.sync_copy(rid_hbm.at[pl.ds(off, 128)], rid_v)
            for k in range(8):
                lin_v[pl.ds(k * 16, 16)] += coff
            pltpu.async_copy(rid_v, out_hbm.at[lin_v], sem).wait()


def kernel(features, coords, W1a, g1, beta1, W1b, W2a, g2, beta2, W2b):
    coords = coords.astype(jnp.int32)
    n = features.shape[0]

    f_pad = jnp.zeros((NPAD, C), jnp.float32).at[:n].set(features)
    z = jnp.full((NPAD,), 1 << 20, jnp.int32).at[:n].set(coords[:, 1])
    y = jnp.zeros((NPAD,), jnp.int32).at[:n].set(coords[:, 2])
    x = jnp.zeros((NPAD,), jnp.int32).at[:n].set(coords[:, 3])
    lin = jnp.full((NPAD,), G, jnp.int32).at[:n].set(
        (coords[:, 1] * HH + coords[:, 2]) * WW + coords[:, 3])
    rid = jnp.arange(NPAD, dtype=jnp.int32)

    grid1 = _make_sc_scatter()(lin, rid)
    idx1, idx2 = _make_sc_build_idx()(grid1, z, y, x)

    p1, p2 = _stage0_mm(f_pad, W1a, W2a)
    h1, h2 = _make_sc_gather_relu()(
        p1.reshape(TAPS * NPAD, C), p2.reshape(TAPS * NPAD, C), idx1, idx2)

    stats = _bn_stats(h1, h2)
    q1, q2 = _stage2_mm(h1, h2, stats, g1, beta1, g2, beta2, W1b, W2b)
    out = _make_sc_gather_sum()(
        q1.reshape(TAPS * NPAD, C), q2.reshape(TAPS * NPAD, C), idx2, idx1)
    return out[:n]


@_ft.lru_cache(maxsize=None)
def _make_sc_build_idx():
  return functools.partial(
    pl.kernel,
    out_type=[
        jax.ShapeDtypeStruct((TAPS * NPAD,), jnp.int32),
        jax.ShapeDtypeStruct((TAPS * NPAD,), jnp.int32),
    ],
    mesh=_sc_mesh(),
    scratch_types=[
        pltpu.VMEM((RPW,), jnp.int32),
        pltpu.VMEM((RPW,), jnp.int32),
        pltpu.VMEM((RPW,), jnp.int32),
        pltpu.VMEM((RPW,), jnp.int32),
        pltpu.VMEM((RPW,), jnp.int32),
        pltpu.VMEM((RPW,), jnp.int32),
        pltpu.SemaphoreType.DMA,
    ],
  )(_sc_build_idx_body)


def _sc_build_idx_body(grid_hbm, z_hbm, y_hbm, x_hbm, idx1_hbm, idx2_hbm,
                  zv, yv, xv, qv, gv, ov, sem):
    coff = lax.axis_index("c") * GPAD
    base = _wid() * RPW
    pltpu.sync_copy(z_hbm.at[pl.ds(base, RPW)], zv)
    pltpu.sync_copy(y_hbm.at[pl.ds(base, RPW)], yv)
    pltpu.sync_copy(x_hbm.at[pl.ds(base, RPW)], xv)

    for t, (dz, dy, dx) in enumerate(_OFF_ALL):
        def qbody(k, _):
            zz = zv[pl.ds(k * 16, 16)] + dz
            yy = yv[pl.ds(k * 16, 16)] + dy
            xx = xv[pl.ds(k * 16, 16)] + dx
            ok = (zz >= 0) & (zz < DD) & (yy >= 0) & (yy < HH)
            ok &= (xx >= 0) & (xx < WW)
            q = (zz * HH + yy) * WW + xx
            qv[pl.ds(k * 16, 16)] = jnp.where(ok, q, G) + coff
            return 0

        lax.fori_loop(0, RPW // 16, qbody, 0, unroll=4)
        for j in range(RPW // 128):
            pltpu.async_copy(
                grid_hbm.at[qv.at[pl.ds(j * 128, 128)]],
                gv.at[pl.ds(j * 128, 128)], sem).wait()

        s1, s2 = _TAP_DEST[t]
        slot = s1 if s1 is not None else s2

        def obody(k, _):
            g = gv[pl.ds(k * 16, 16)]
            ov[pl.ds(k * 16, 16)] = jnp.where(
                g < NN, g + slot * NPAD, -1)
            return 0

        lax.fori_loop(0, RPW // 16, obody, 0, unroll=4)
        if s1 is not None:
            pltpu.sync_copy(ov, idx1_hbm.at[pl.ds(s1 * NPAD + base, RPW)])
        if s2 is not None:
            pltpu.sync_copy(ov, idx2_hbm.at[pl.ds(s2 * NPAD + base, RPW)])


NBUF = 4


def _accum_taps(bufset, taps, acc, relu, add_in):
    first = taps[0] == 0
    last = taps[-1] == TAPS - 1

    def abody(r, _):
        for g in range(C // 16):
            sl = pl.ds(g * 16, 16)
            v = bufset[0][r, sl]
            if len(taps) > 1:
                v = v + bufset[1][r, sl]
            if not first or (first and add_in):
                v = v + acc[r, sl]
            if relu and last:
                v = jnp.maximum(v, 0.0)
            acc[r, sl] = v
        return 0

    lax.fori_loop(0, CHUNK, abody, 0)


_UNIT_TAPS = [(0, 1), (2, 3), (4, 5), (6, 7), (8,)]


def _run_gather_jobs(jobs, bufs, acc, idx_refs, sem):
    units = [(jb, taps) for jb in jobs for taps in _UNIT_TAPS]

    def zero(buf):
        def zbody(r, _):
            for g in range(C // 16):
                buf[r, pl.ds(g * 16, 16)] = jnp.zeros((16,), jnp.float32)
            return 0

        lax.fori_loop(0, CHUNK, zbody, 0)

    def fire(u):
        jb, taps = units[u]
        tab, ioff, out_row, _, _ = jb
        bset = bufs[2 * (u % 2):2 * (u % 2) + 2]
        iv = idx_refs[jb[4]]
        cps = []
        for k, t in enumerate(taps):
            if t == TAPS // 2:
                cps.append(pltpu.async_copy(
                    tab.at[pl.ds((TAPS // 2) * NPAD + out_row, CHUNK), :],
                    bset[k], sem))
            else:
                zero(bset[k])
                cps.append(pltpu.async_copy(
                    tab.at[plsc.Indices(
                        iv.at[pl.ds(ioff + t * RPW, CHUNK)],
                        ignored_value=-1)],
                    bset[k], sem))
        return cps

    cps = {u: fire(u) for u in range(2)}
    for u, (jb, taps) in enumerate(units):
        for cp in cps.pop(u):
            cp.wait()
        tab, ioff, out_row, (out_hbm, relu, add_in), _ = jb
        bset = bufs[2 * (u % 2):2 * (u % 2) + 2]
        _accum_taps(bset, taps, acc, relu, add_in)
        if taps[-1] == TAPS - 1 and out_hbm is not None:
            pltpu.sync_copy(acc, out_hbm.at[pl.ds(out_row, CHUNK), :])
        if u + 2 < len(units):
            cps[u + 2] = fire(u + 2)


_GATHER_SCRATCH = (
    [pltpu.VMEM((CHUNK, C), jnp.float32) for _ in range(NBUF)]
    + [
        pltpu.VMEM((CHUNK, C), jnp.float32),
        pltpu.VMEM((2 * TAPS * RPW,), jnp.int32),
        pltpu.SemaphoreType.DMA,
    ]
)


@_ft.lru_cache(maxsize=None)
def _make_sc_gather_relu():
  return functools.partial(
    pl.kernel,
    out_type=[
        jax.ShapeDtypeStruct((NPAD, C), jnp.float32),
        jax.ShapeDtypeStruct((NPAD, C), jnp.float32),
    ],
    mesh=_sc_mesh(),
    scratch_types=_GATHER_SCRATCH,
  )(_sc_gather_relu_body)


def _sc_gather_relu_body(p1_hbm, p2_hbm, idx1_hbm, idx2_hbm, h1_hbm, h2_hbm,
                         *scratch):
    bufs = list(scratch[:NBUF])
    acc = scratch[NBUF]
    idx_v = scratch[NBUF + 1]
    sem = scratch[NBUF + 2]
    base = _wid() * RPW
    for i in range(TAPS):
        pltpu.sync_copy(idx1_hbm.at[pl.ds(i * NPAD + base, RPW)],
                        idx_v.at[pl.ds(i * RPW, RPW)])
        pltpu.sync_copy(idx2_hbm.at[pl.ds(i * NPAD + base, RPW)],
                        idx_v.at[pl.ds((TAPS + i) * RPW, RPW)])
    jobs = [
        (p1_hbm, ch * CHUNK, base + ch * CHUNK, (h1_hbm, True, False), 0)
        for ch in range(NCHUNK)
    ] + [
        (p2_hbm, TAPS * RPW + ch * CHUNK, base + ch * CHUNK,
         (h2_hbm, True, False), 0)
        for ch in range(NCHUNK)
    ]
    _run_gather_jobs(jobs, bufs, acc, [idx_v], sem)


_SUM2_SCRATCH = (
    [pltpu.VMEM((CHUNK, C), jnp.float32) for _ in range(NBUF)]
    + [
        pltpu.VMEM((CHUNK, C), jnp.float32),
        pltpu.VMEM((TAPS * RPW,), jnp.int32),
        pltpu.VMEM((TAPS * RPW,), jnp.int32),
        pltpu.SemaphoreType.DMA,
    ]
)


@_ft.lru_cache(maxsize=None)
def _make_sc_gather_sum():
  return functools.partial(
    pl.kernel,
    out_type=jax.ShapeDtypeStruct((NPAD, C), jnp.float32),
    mesh=_sc_mesh(),
    scratch_types=_SUM2_SCRATCH,
  )(_sc_gather_sum_body)


def _sc_gather_sum_body(q1_hbm, q2_hbm, idxa_hbm, idxb_hbm, out_hbm, *scratch):
    bufs = list(scratch[:NBUF])
    acc = scratch[NBUF]
    ia_v = scratch[NBUF + 1]
    ib_v = scratch[NBUF + 2]
    sem = scratch[NBUF + 3]
    base = _wid() * RPW
    for i in range(TAPS):
        pltpu.sync_copy(idxa_hbm.at[pl.ds(i * NPAD + base, RPW)],
                        ia_v.at[pl.ds(i * RPW, RPW)])
        pltpu.sync_copy(idxb_hbm.at[pl.ds(i * NPAD + base, RPW)],
                        ib_v.at[pl.ds(i * RPW, RPW)])
    jobs = []
    for ch in range(NCHUNK):
        jobs.append((q1_hbm, ch * CHUNK, base + ch * CHUNK,
                     (None, False, False), 0))
        jobs.append((q2_hbm, ch * CHUNK, base + ch * CHUNK,
                     (out_hbm, False, True), 1))
    _run_gather_jobs(jobs, bufs, acc, [ia_v, ib_v], sem)


# speedup vs baseline: 1.0844x; 1.0844x over previous
import functools

import jax
import jax.numpy as jnp
from jax import lax
from jax.experimental import pallas as pl
from jax.experimental.pallas import tpu as pltpu
from jax.experimental.pallas import tpu_sc as plsc

DD, HH, WW = 64, 256, 256
NN = 20000
C = 128
TAPS = 9

NC, NS = 2, 16
NW = NC * NS
NPAD = 20480
RPW = NPAD // NW
CHUNK = 128
NCHUNK = RPW // CHUNK

G = DD * HH * WW
GROWS = 16512
GPAD = GROWS * 256
SENT = NN

_OFF_Z1X = [(dz, 0, dx) for dz in (-1, 0, 1) for dx in (-1, 0, 1)]
_OFF_1YX = [(0, dy, dx) for dy in (-1, 0, 1) for dx in (-1, 0, 1)]
_OFF_ALL = _OFF_Z1X[:3] + _OFF_Z1X[3:6] + _OFF_Z1X[6:] + _OFF_1YX[:3] + _OFF_1YX[6:]
_TAP_DEST = (
    [(i, None) for i in range(3)]
    + [(i, i) for i in range(3, 6)]
    + [(i, None) for i in range(6, 9)]
    + [(None, j) for j in range(3)]
    + [(None, j) for j in range(6, 9)]
)

import functools as _ft


@_ft.lru_cache(maxsize=None)
def _sc_mesh():
    return plsc.VectorSubcoreMesh(
        core_axis_name="c", subcore_axis_name="s",
        num_cores=NC, num_subcores=NS)


def _wid():
    return lax.axis_index("s") * NC + lax.axis_index("c")


def _mm_body(x_ref, wa_ref, wb_ref, oa_ref, ob_ref):
    x = x_ref[...].astype(jnp.bfloat16)
    oa_ref[0] = jnp.dot(x, wa_ref[0].astype(jnp.bfloat16),
                        preferred_element_type=jnp.float32)
    ob_ref[0] = jnp.dot(x, wb_ref[0].astype(jnp.bfloat16),
                        preferred_element_type=jnp.float32)


def _stage0_mm(f_pad, w1, w2):
    bm = 256
    return pl.pallas_call(
        _mm_body,
        grid=(NPAD // bm, TAPS),
        in_specs=[
            pl.BlockSpec((bm, C), lambda i, j: (i, 0)),
            pl.BlockSpec((1, C, C), lambda i, j: (j, 0, 0)),
            pl.BlockSpec((1, C, C), lambda i, j: (j, 0, 0)),
        ],
        out_specs=[
            pl.BlockSpec((1, bm, C), lambda i, j: (j, i, 0)),
            pl.BlockSpec((1, bm, C), lambda i, j: (j, i, 0)),
        ],
        out_shape=[
            jax.ShapeDtypeStruct((TAPS, NPAD, C), jnp.float32),
            jax.ShapeDtypeStruct((TAPS, NPAD, C), jnp.float32),
        ],
    )(f_pad, w1, w2)


def _stats_body(h1_ref, h2_ref, out_ref):
    @pl.when(pl.program_id(0) == 0)
    def _init():
        out_ref[...] = jnp.zeros((8, C), jnp.float32)

    h1 = h1_ref[...]
    h2 = h2_ref[...]
    upd = jnp.concatenate([
        jnp.sum(h1, axis=0)[None, :],
        jnp.sum(h1 * h1, axis=0)[None, :],
        jnp.sum(h2, axis=0)[None, :],
        jnp.sum(h2 * h2, axis=0)[None, :],
        jnp.zeros((4, C), jnp.float32),
    ], axis=0)
    out_ref[...] += upd


def _bn_stats(h1, h2):
    bm = 2048
    return pl.pallas_call(
        _stats_body,
        grid=(NPAD // bm,),
        in_specs=[
            pl.BlockSpec((bm, C), lambda i: (i, 0)),
            pl.BlockSpec((bm, C), lambda i: (i, 0)),
        ],
        out_specs=pl.BlockSpec((8, C), lambda i: (0, 0)),
        out_shape=jax.ShapeDtypeStruct((8, C), jnp.float32),
    )(h1, h2)


def _mm2_body(h1_ref, h2_ref, st_ref, g1_ref, b1_ref, g2_ref, b2_ref,
              w1_ref, w2_ref, oa_ref, ob_ref):
    bm = h1_ref.shape[0]
    eps = 1e-5
    n = jnp.float32(NN)
    row = pl.program_id(0) * bm + lax.broadcasted_iota(jnp.int32, (bm, 1), 0)
    live = (row < NN).astype(jnp.float32)

    def bn(h, srow, g, b):
        m = st_ref[srow] / n
        v = st_ref[srow + 1] / n - m * m
        s = g * lax.rsqrt(v + eps)
        t = b - m * s
        return (h * s[None, :] + t[None, :]) * live

    hb1 = bn(h1_ref[...], 0, g1_ref[...], b1_ref[...])
    hb2 = bn(h2_ref[...], 2, g2_ref[...], b2_ref[...])
    oa_ref[0] = jnp.dot(hb1.astype(jnp.bfloat16),
                        w1_ref[0].astype(jnp.bfloat16),
                        preferred_element_type=jnp.float32)
    ob_ref[0] = jnp.dot(hb2.astype(jnp.bfloat16),
                        w2_ref[0].astype(jnp.bfloat16),
                        preferred_element_type=jnp.float32)


def _stage2_mm(h1, h2, stats, g1, b1, g2, b2, w1, w2):
    bm = 256
    vec = lambda: pl.BlockSpec((C,), lambda i, j: (0,))
    return pl.pallas_call(
        _mm2_body,
        grid=(NPAD // bm, TAPS),
        in_specs=[
            pl.BlockSpec((bm, C), lambda i, j: (i, 0)),
            pl.BlockSpec((bm, C), lambda i, j: (i, 0)),
            pl.BlockSpec((8, C), lambda i, j: (0, 0)),
            vec(), vec(), vec(), vec(),
            pl.BlockSpec((1, C, C), lambda i, j: (j, 0, 0)),
            pl.BlockSpec((1, C, C), lambda i, j: (j, 0, 0)),
        ],
        out_specs=[
            pl.BlockSpec((1, bm, C), lambda i, j: (j, i, 0)),
            pl.BlockSpec((1, bm, C), lambda i, j: (j, i, 0)),
        ],
        out_shape=[
            jax.ShapeDtypeStruct((TAPS, NPAD, C), jnp.float32),
            jax.ShapeDtypeStruct((TAPS, NPAD, C), jnp.float32),
        ],
    )(h1, h2, stats, g1, b1, g2, b2, w1, w2)


_SLAB = GPAD // NS
_SLAB_CH = 16512
_VROWS = 16
_NACT = (NPAD // 128) // _VROWS


@_ft.lru_cache(maxsize=None)
def _make_sc_grid_idx():
  return functools.partial(
    pl.kernel,
    out_type=[
        jax.ShapeDtypeStruct((NC * GPAD,), jnp.int32),
        jax.ShapeDtypeStruct((TAPS * NPAD,), jnp.int32),
        jax.ShapeDtypeStruct((TAPS * NPAD,), jnp.int32),
    ],
    mesh=_sc_mesh(),
    scratch_types=[
        pltpu.VMEM((128,), jnp.int32),
        pltpu.VMEM((128,), jnp.int32),
        pltpu.VMEM((_SLAB_CH,), jnp.int32),
        pltpu.VMEM((RPW,), jnp.int32),
        pltpu.VMEM((RPW,), jnp.int32),
        pltpu.VMEM((RPW,), jnp.int32),
        pltpu.VMEM((RPW,), jnp.int32),
        pltpu.VMEM((RPW,), jnp.int32),
        pltpu.VMEM((RPW,), jnp.int32),
        pltpu.SemaphoreType.DMA,
    ],
  )(_sc_grid_idx_body)


def _sc_grid_idx_body(lin_hbm, rid_hbm, z_hbm, y_hbm, x_hbm,
                      grid_hbm, idx1_hbm, idx2_hbm,
                      lin_v, rid_v, fill, zv, yv, xv, qv, gv, ov, sem):
    c = lax.axis_index("c")
    s = lax.axis_index("s")
    coff = c * GPAD

    def fbody(k, _):
        fill[pl.ds(k * 16, 16)] = jnp.full((16,), SENT, jnp.int32)
        return 0

    lax.fori_loop(0, _SLAB_CH // 16, fbody, 0, unroll=8)
    gb = coff + s * _SLAB
    for j in range(_SLAB // _SLAB_CH):
        pltpu.sync_copy(fill, grid_hbm.at[pl.ds(gb + j * _SLAB_CH, _SLAB_CH)])
    plsc.subcore_barrier()

    @pl.when(s < _NACT)
    def _do_scatter():
        vb = s * (_VROWS * 128)
        for j in range(_VROWS):
            off = vb + j * 128
            pltpu.sync_copy(lin_hbm.at[pl.ds(off, 128)], lin_v)
            pltpu.sync_copy(rid_hbm.at[pl.ds(off, 128)], rid_v)
            for k in range(8):
                lin_v[pl.ds(k * 16, 16)] += coff
            pltpu.async_copy(rid_v, grid_hbm.at[lin_v], sem).wait()

    plsc.subcore_barrier()

    base = _wid() * RPW
    pltpu.sync_copy(z_hbm.at[pl.ds(base, RPW)], zv)
    pltpu.sync_copy(y_hbm.at[pl.ds(base, RPW)], yv)
    pltpu.sync_copy(x_hbm.at[pl.ds(base, RPW)], xv)

    for t, (dz, dy, dx) in enumerate(_OFF_ALL):
        def qbody(k, _):
            zz = zv[pl.ds(k * 16, 16)] + dz
            yy = yv[pl.ds(k * 16, 16)] + dy
            xx = xv[pl.ds(k * 16, 16)] + dx
            ok = (zz >= 0) & (zz < DD) & (yy >= 0) & (yy < HH)
            ok &= (xx >= 0) & (xx < WW)
            q = (zz * HH + yy) * WW + xx
            qv[pl.ds(k * 16, 16)] = jnp.where(ok, q, G) + coff
            return 0

        lax.fori_loop(0, RPW // 16, qbody, 0, unroll=4)
        for j in range(RPW // 128):
            pltpu.async_copy(
                grid_hbm.at[qv.at[pl.ds(j * 128, 128)]],
                gv.at[pl.ds(j * 128, 128)], sem).wait()

        s1, s2 = _TAP_DEST[t]
        slot = s1 if s1 is not None else s2

        def obody(k, _):
            g = gv[pl.ds(k * 16, 16)]
            ov[pl.ds(k * 16, 16)] = jnp.where(
                g < NN, g + slot * NPAD, -1)
            return 0

        lax.fori_loop(0, RPW // 16, obody, 0, unroll=4)
        if s1 is not None:
            pltpu.sync_copy(ov, idx1_hbm.at[pl.ds(s1 * NPAD + base, RPW)])
        if s2 is not None:
            pltpu.sync_copy(ov, idx2_hbm.at[pl.ds(s2 * NPAD + base, RPW)])


def kernel(features, coords, W1a, g1, beta1, W1b, W2a, g2, beta2, W2b):
    coords = coords.astype(jnp.int32)
    n = features.shape[0]

    f_pad = jnp.zeros((NPAD, C), jnp.float32).at[:n].set(features)
    z = jnp.full((NPAD,), 1 << 20, jnp.int32).at[:n].set(coords[:, 1])
    y = jnp.zeros((NPAD,), jnp.int32).at[:n].set(coords[:, 2])
    x = jnp.zeros((NPAD,), jnp.int32).at[:n].set(coords[:, 3])
    lin = jnp.full((NPAD,), G, jnp.int32).at[:n].set(
        (coords[:, 1] * HH + coords[:, 2]) * WW + coords[:, 3])
    rid = jnp.arange(NPAD, dtype=jnp.int32)

    _, idx1, idx2 = _make_sc_grid_idx()(lin, rid, z, y, x)

    p1, p2 = _stage0_mm(f_pad, W1a, W2a)
    h1, h2 = _make_sc_gather_relu()(
        p1.reshape(TAPS * NPAD, C), p2.reshape(TAPS * NPAD, C), idx1, idx2)

    stats = _bn_stats(h1, h2)
    q1, q2 = _stage2_mm(h1, h2, stats, g1, beta1, g2, beta2, W1b, W2b)
    out = _make_sc_gather_sum()(
        q1.reshape(TAPS * NPAD, C), q2.reshape(TAPS * NPAD, C), idx2, idx1)
    return out[:n]


NBUF = 4


def _accum_taps(bufset, taps, acc, relu, add_in):
    first = taps[0] == 0
    last = taps[-1] == TAPS - 1

    def abody(r, _):
        for g in range(C // 16):
            sl = pl.ds(g * 16, 16)
            v = bufset[0][r, sl]
            if len(taps) > 1:
                v = v + bufset[1][r, sl]
            if not first or (first and add_in):
                v = v + acc[r, sl]
            if relu and last:
                v = jnp.maximum(v, 0.0)
            acc[r, sl] = v
        return 0

    lax.fori_loop(0, CHUNK, abody, 0)


_UNIT_TAPS = [(0, 1), (2, 3), (4, 5), (6, 7), (8,)]


def _run_gather_jobs(jobs, bufs, acc, idx_refs, sem):
    units = [(jb, taps) for jb in jobs for taps in _UNIT_TAPS]

    def zero(buf):
        def zbody(r, _):
            for g in range(C // 16):
                buf[r, pl.ds(g * 16, 16)] = jnp.zeros((16,), jnp.float32)
            return 0

        lax.fori_loop(0, CHUNK, zbody, 0)

    def fire(u):
        jb, taps = units[u]
        tab, ioff, out_row, _, _ = jb
        bset = bufs[2 * (u % 2):2 * (u % 2) + 2]
        iv = idx_refs[jb[4]]
        cps = []
        for k, t in enumerate(taps):
            if t == TAPS // 2:
                cps.append(pltpu.async_copy(
                    tab.at[pl.ds((TAPS // 2) * NPAD + out_row, CHUNK), :],
                    bset[k], sem))
            else:
                zero(bset[k])
                cps.append(pltpu.async_copy(
                    tab.at[plsc.Indices(
                        iv.at[pl.ds(ioff + t * RPW, CHUNK)],
                        ignored_value=-1)],
                    bset[k], sem))
        return cps

    cps = {u: fire(u) for u in range(2)}
    for u, (jb, taps) in enumerate(units):
        for cp in cps.pop(u):
            cp.wait()
        tab, ioff, out_row, (out_hbm, relu, add_in), _ = jb
        bset = bufs[2 * (u % 2):2 * (u % 2) + 2]
        _accum_taps(bset, taps, acc, relu, add_in)
        if taps[-1] == TAPS - 1 and out_hbm is not None:
            pltpu.sync_copy(acc, out_hbm.at[pl.ds(out_row, CHUNK), :])
        if u + 2 < len(units):
            cps[u + 2] = fire(u + 2)


_GATHER_SCRATCH = (
    [pltpu.VMEM((CHUNK, C), jnp.float32) for _ in range(NBUF)]
    + [
        pltpu.VMEM((CHUNK, C), jnp.float32),
        pltpu.VMEM((2 * TAPS * RPW,), jnp.int32),
        pltpu.SemaphoreType.DMA,
    ]
)


@_ft.lru_cache(maxsize=None)
def _make_sc_gather_relu():
  return functools.partial(
    pl.kernel,
    out_type=[
        jax.ShapeDtypeStruct((NPAD, C), jnp.float32),
        jax.ShapeDtypeStruct((NPAD, C), jnp.float32),
    ],
    mesh=_sc_mesh(),
    scratch_types=_GATHER_SCRATCH,
  )(_sc_gather_relu_body)


def _sc_gather_relu_body(p1_hbm, p2_hbm, idx1_hbm, idx2_hbm, h1_hbm, h2_hbm,
                         *scratch):
    bufs = list(scratch[:NBUF])
    acc = scratch[NBUF]
    idx_v = scratch[NBUF + 1]
    sem = scratch[NBUF + 2]
    base = _wid() * RPW
    for i in range(TAPS):
        pltpu.sync_copy(idx1_hbm.at[pl.ds(i * NPAD + base, RPW)],
                        idx_v.at[pl.ds(i * RPW, RPW)])
        pltpu.sync_copy(idx2_hbm.at[pl.ds(i * NPAD + base, RPW)],
                        idx_v.at[pl.ds((TAPS + i) * RPW, RPW)])
    jobs = [
        (p1_hbm, ch * CHUNK, base + ch * CHUNK, (h1_hbm, True, False), 0)
        for ch in range(NCHUNK)
    ] + [
        (p2_hbm, TAPS * RPW + ch * CHUNK, base + ch * CHUNK,
         (h2_hbm, True, False), 0)
        for ch in range(NCHUNK)
    ]
    _run_gather_jobs(jobs, bufs, acc, [idx_v], sem)


_SUM2_SCRATCH = (
    [pltpu.VMEM((CHUNK, C), jnp.float32) for _ in range(NBUF)]
    + [
        pltpu.VMEM((CHUNK, C), jnp.float32),
        pltpu.VMEM((TAPS * RPW,), jnp.int32),
        pltpu.VMEM((TAPS * RPW,), jnp.int32),
        pltpu.SemaphoreType.DMA,
    ]
)


@_ft.lru_cache(maxsize=None)
def _make_sc_gather_sum():
  return functools.partial(
    pl.kernel,
    out_type=jax.ShapeDtypeStruct((NPAD, C), jnp.float32),
    mesh=_sc_mesh(),
    scratch_types=_SUM2_SCRATCH,
  )(_sc_gather_sum_body)


def _sc_gather_sum_body(q1_hbm, q2_hbm, idxa_hbm, idxb_hbm, out_hbm, *scratch):
    bufs = list(scratch[:NBUF])
    acc = scratch[NBUF]
    ia_v = scratch[NBUF + 1]
    ib_v = scratch[NBUF + 2]
    sem = scratch[NBUF + 3]
    base = _wid() * RPW
    for i in range(TAPS):
        pltpu.sync_copy(idxa_hbm.at[pl.ds(i * NPAD + base, RPW)],
                        ia_v.at[pl.ds(i * RPW, RPW)])
        pltpu.sync_copy(idxb_hbm.at[pl.ds(i * NPAD + base, RPW)],
                        ib_v.at[pl.ds(i * RPW, RPW)])
    jobs = []
    for ch in range(NCHUNK):
        jobs.append((q1_hbm, ch * CHUNK, base + ch * CHUNK,
                     (None, False, False), 0))
        jobs.append((q2_hbm, ch * CHUNK, base + ch * CHUNK,
                     (out_hbm, False, True), 1))
    _run_gather_jobs(jobs, bufs, acc, [ia_v, ib_v], sem)
